# baseline (speedup 1.0000x reference)
import functools

import jax
import jax.numpy as jnp
from jax import lax
from jax.experimental import pallas as pl
from jax.experimental.pallas import tpu as pltpu
from jax.experimental.pallas import tpu_sc as plsc

NC = 2
NS = 16
NW = NC * NS
L = 16

BATCH = 16384
K = 32
BPW = BATCH // NW
HALF = BPW // 2
CHUNK = 128
NCHUNK = BPW // CHUNK


def _qgather_body(iid_hbm, q_hbm, qrows_hbm, iidx_v, qi_v, sem):
    wid = lax.axis_index("s") * NC + lax.axis_index("c")
    base = wid * BPW

    for j in range(NCHUNK):
        pltpu.sync_copy(iid_hbm.at[pl.ds(base + j * CHUNK, CHUNK)],
                        iidx_v.at[j])

    copies = []
    for j in range(NCHUNK):
        copies.append(pltpu.async_copy(
            q_hbm.at[iidx_v.at[j]], qi_v.at[pl.ds(j * CHUNK, CHUNK)], sem))
    for c in copies:
        c.wait()

    pltpu.sync_copy(qi_v, qrows_hbm.at[pl.ds(base, BPW)])


def _pdot_body(uid_hbm, p_hbm, qrows_hbm, out_hbm,
               uid_v, pu_v, qi_v, o_v, sem):
    wid = lax.axis_index("s") * NC + lax.axis_index("c")
    base = wid * BPW

    pltpu.sync_copy(uid_hbm.at[pl.ds(base, BPW)], uid_v)

    lane = lax.iota(jnp.int32, L)

    def half(h, carry):
        hbase = h * HALF

        pltpu.sync_copy(
            qrows_hbm.at[pl.ds(base + hbase, HALF)], qi_v)

        def fire(g, c):
            uvec = uid_v[pl.ds(hbase + g * L, L)]
            for j in range(L):
                pltpu.async_copy(p_hbm.at[uvec[j]], pu_v.at[g * L + j], sem)
            return c

        lax.fori_loop(0, HALF // L, fire, 0)

        def drain(b, c):
            pltpu.make_async_copy(p_hbm.at[0], pu_v.at[b], sem).wait()
            return c

        lax.fori_loop(0, HALF, drain, 0)

        def group(g, c):
            row = g * L + lane

            def col_step(t, acc):
                col = (lane + t) & (K - 1)
                pv = plsc.load_gather(pu_v, [row, col])
                qv = plsc.load_gather(qi_v, [row, col])
                return acc + pv * qv

            acc = lax.fori_loop(0, K, col_step, jnp.zeros((L,), jnp.float32))
            o_v[pl.ds(hbase + g * L, L)] = acc
            return c

        lax.fori_loop(0, HALF // L, group, 0)
        return carry

    lax.fori_loop(0, BPW // HALF, half, 0)

    pltpu.sync_copy(o_v, out_hbm.at[pl.ds(base, BPW)])


def _gmf(user_ids, item_ids, P, Q):
    mesh = plsc.VectorSubcoreMesh(
        core_axis_name="c", subcore_axis_name="s",
        num_cores=NC, num_subcores=NS)

    qgather = pl.kernel(
        _qgather_body,
        out_type=jax.ShapeDtypeStruct((BATCH, K), jnp.float32),
        mesh=mesh,
        compiler_params=pltpu.CompilerParams(
            needs_layout_passes=False, use_tc_tiling_on_sc=False),
        scratch_types=[
            pltpu.VMEM((NCHUNK, CHUNK), jnp.int32),
            pltpu.VMEM((BPW, K), jnp.float32),
            pltpu.SemaphoreType.DMA,
        ],
    )
    qrows = qgather(item_ids, Q)

    pdot = pl.kernel(
        _pdot_body,
        out_type=jax.ShapeDtypeStruct((BATCH,), jnp.float32),
        mesh=mesh,
        compiler_params=pltpu.CompilerParams(needs_layout_passes=False),
        scratch_types=[
            pltpu.VMEM((BPW,), jnp.int32),
            pltpu.VMEM((HALF, K), jnp.float32),
            pltpu.VMEM((HALF, K), jnp.float32),
            pltpu.VMEM((BPW,), jnp.float32),
            pltpu.SemaphoreType.DMA,
        ],
    )
    return pdot(user_ids, P, qrows)


def kernel(user_ids, item_ids, P, Q):
    out = _gmf(user_ids, item_ids, P, Q)
    return out.reshape(BATCH, 1)
